# baseline (speedup 1.0000x reference)
import jax
import jax.numpy as jnp
from jax.experimental import pallas as pl
from jax.experimental.pallas import tpu as pltpu

N = 2048
D_EMB = 256
K = 30
ALPHA = 3.0
H = 64
ROWS = 128

_NT = (((1,), (1,)), ((), ()))


def _fused_body(m1_ref, m2_ref, m1f_ref, m2f_ref, x_ref, w1_ref, w2_ref,
                adj_ref, out_ref, t_ref):
    m1 = m1_ref[...]
    m2 = m2_ref[...]
    logits = jax.lax.dot_general(m1, m2f_ref[...], _NT,
                                 preferred_element_type=jnp.float32)
    logits = logits - jax.lax.dot_general(m2, m1f_ref[...], _NT,
                                          preferred_element_type=jnp.float32)
    theta = jnp.maximum(jnp.tanh(ALPHA * logits), 0.0)

    neg = jnp.float32(-3e38)
    t = jnp.max(logits, axis=1, keepdims=True)
    for _ in range(K - 1):
        t = jnp.max(jnp.where(logits < t, logits, neg), axis=1,
                    keepdims=True)
    t_ref[...] = jnp.broadcast_to(t, t_ref.shape)

    cnt = jnp.sum(jnp.where(logits >= t, 1.0, 0.0), axis=1, keepdims=True)

    @pl.when(jnp.any(cnt > jnp.float32(K)))
    def _guarded_walk():
        tg = jnp.max(logits, axis=1, keepdims=True)
        for _ in range(K - 1):
            c = jnp.sum(jnp.where(logits >= tg, 1.0, 0.0), axis=1,
                        keepdims=True)
            nxt = jnp.max(jnp.where(logits < tg, logits, neg), axis=1,
                          keepdims=True)
            tg = jnp.where(c < jnp.float32(K), nxt, tg)
        t_ref[...] = jnp.broadcast_to(tg, t_ref.shape)

    adj = jnp.where(logits >= t_ref[:, :1], theta, 0.0)
    adj_ref[...] = adj

    deg = jnp.sum(adj, axis=1, keepdims=True) + 1e-8
    an = adj / deg
    prop = jnp.dot(an, x_ref[...], preferred_element_type=jnp.float32)

    bt = prop.shape[1] // 2
    p0 = prop[:, :bt]
    p1 = prop[:, bt:]
    acc0 = jnp.zeros_like(p0)
    acc1 = jnp.zeros_like(p1)
    for h in range(H):
        hh = jnp.tanh(p0 * w1_ref[0:1, h:h + 1] + p1 * w1_ref[1:2, h:h + 1])
        acc0 = acc0 + hh * w2_ref[h:h + 1, 0:1]
        acc1 = acc1 + hh * w2_ref[h:h + 1, 1:2]
    out_ref[:, :bt] = acc0
    out_ref[:, bt:] = acc1


def kernel(inputs, targets, entire_inputs, edge_index, M1, M2, W1, W2):
    B, T, n, F = inputs.shape
    BT = B * T
    X = jnp.transpose(inputs, (2, 3, 0, 1)).reshape(n, F * BT)

    adj, out = pl.pallas_call(
        _fused_body,
        grid=(N // ROWS,),
        in_specs=[
            pl.BlockSpec((ROWS, D_EMB), lambda i: (i, 0)),
            pl.BlockSpec((ROWS, D_EMB), lambda i: (i, 0)),
            pl.BlockSpec((N, D_EMB), lambda i: (0, 0)),
            pl.BlockSpec((N, D_EMB), lambda i: (0, 0)),
            pl.BlockSpec((N, F * BT), lambda i: (0, 0)),
            pl.BlockSpec((2, H), lambda i: (0, 0)),
            pl.BlockSpec((H, 2), lambda i: (0, 0)),
        ],
        out_specs=[
            pl.BlockSpec((ROWS, N), lambda i: (i, 0)),
            pl.BlockSpec((ROWS, F * BT), lambda i: (i, 0)),
        ],
        out_shape=[
            jax.ShapeDtypeStruct((N, N), jnp.float32),
            jax.ShapeDtypeStruct((N, F * BT), jnp.float32),
        ],
        scratch_shapes=[pltpu.VMEM((ROWS, 128), jnp.float32)],
    )(M1, M2, M1, M2, X, W1, W2)

    outputs = out.reshape(n, F, B, T).transpose(2, 3, 0, 1)
    return (adj, outputs)


# speedup vs baseline: 1.0541x; 1.0541x over previous
import jax
import jax.numpy as jnp
from jax.experimental import pallas as pl
from jax.experimental.pallas import tpu as pltpu

N = 2048
D_EMB = 256
K = 30
ALPHA = 3.0
H = 64
ROWS = 256

_NT = (((1,), (1,)), ((), ()))


def _fused_body(m1_ref, m2_ref, m1f_ref, m2f_ref, x_ref, w1_ref, w2_ref,
                adj_ref, out_ref, t_ref):
    m1 = m1_ref[...]
    m2 = m2_ref[...]
    logits = jax.lax.dot_general(m1, m2f_ref[...], _NT,
                                 preferred_element_type=jnp.float32)
    logits = logits - jax.lax.dot_general(m2, m1f_ref[...], _NT,
                                          preferred_element_type=jnp.float32)
    theta = jnp.maximum(jnp.tanh(ALPHA * logits), 0.0)

    neg = jnp.float32(-3e38)
    t = jnp.max(logits, axis=1, keepdims=True)
    for _ in range(K - 1):
        t = jnp.max(jnp.where(logits < t, logits, neg), axis=1,
                    keepdims=True)
    t_ref[...] = jnp.broadcast_to(t, t_ref.shape)

    cnt = jnp.sum(jnp.where(logits >= t, 1.0, 0.0), axis=1, keepdims=True)

    @pl.when(jnp.any(cnt > jnp.float32(K)))
    def _guarded_walk():
        tg = jnp.max(logits, axis=1, keepdims=True)
        for _ in range(K - 1):
            c = jnp.sum(jnp.where(logits >= tg, 1.0, 0.0), axis=1,
                        keepdims=True)
            nxt = jnp.max(jnp.where(logits < tg, logits, neg), axis=1,
                          keepdims=True)
            tg = jnp.where(c < jnp.float32(K), nxt, tg)
        t_ref[...] = jnp.broadcast_to(tg, t_ref.shape)

    adj = jnp.where(logits >= t_ref[:, :1], theta, 0.0)
    adj_ref[...] = adj

    deg = jnp.sum(adj, axis=1, keepdims=True) + 1e-8
    an = adj / deg
    prop = jnp.dot(an, x_ref[...], preferred_element_type=jnp.float32)

    bt = prop.shape[1] // 2
    p0 = prop[:, :bt]
    p1 = prop[:, bt:]
    acc0 = jnp.zeros_like(p0)
    acc1 = jnp.zeros_like(p1)
    for h in range(H):
        hh = jnp.tanh(p0 * w1_ref[0:1, h:h + 1] + p1 * w1_ref[1:2, h:h + 1])
        acc0 = acc0 + hh * w2_ref[h:h + 1, 0:1]
        acc1 = acc1 + hh * w2_ref[h:h + 1, 1:2]
    out_ref[:, :bt] = acc0
    out_ref[:, bt:] = acc1


def kernel(inputs, targets, entire_inputs, edge_index, M1, M2, W1, W2):
    B, T, n, F = inputs.shape
    BT = B * T
    X = jnp.transpose(inputs, (2, 3, 0, 1)).reshape(n, F * BT)

    adj, out = pl.pallas_call(
        _fused_body,
        grid=(N // ROWS,),
        in_specs=[
            pl.BlockSpec((ROWS, D_EMB), lambda i: (i, 0)),
            pl.BlockSpec((ROWS, D_EMB), lambda i: (i, 0)),
            pl.BlockSpec((N, D_EMB), lambda i: (0, 0)),
            pl.BlockSpec((N, D_EMB), lambda i: (0, 0)),
            pl.BlockSpec((N, F * BT), lambda i: (0, 0)),
            pl.BlockSpec((2, H), lambda i: (0, 0)),
            pl.BlockSpec((H, 2), lambda i: (0, 0)),
        ],
        out_specs=[
            pl.BlockSpec((ROWS, N), lambda i: (i, 0)),
            pl.BlockSpec((ROWS, F * BT), lambda i: (i, 0)),
        ],
        out_shape=[
            jax.ShapeDtypeStruct((N, N), jnp.float32),
            jax.ShapeDtypeStruct((N, F * BT), jnp.float32),
        ],
        scratch_shapes=[pltpu.VMEM((ROWS, 128), jnp.float32)],
    )(M1, M2, M1, M2, X, W1, W2)

    outputs = out.reshape(n, F, B, T).transpose(2, 3, 0, 1)
    return (adj, outputs)
